# baseline (speedup 1.0000x reference)
import functools

import jax
import jax.numpy as jnp
from jax import lax
from jax.experimental import pallas as pl
from jax.experimental.pallas import tpu as pltpu
from jax.experimental.pallas import tpu_sc as plsc

NC = 2
NS = 16
LANES = 16
NW = NC * NS


def _make_relayout_kernel(dim, vocab):
    nblk = -(-vocab // 128)
    vregs = 128 * dim // LANES

    mesh = plsc.VectorSubcoreMesh(core_axis_name="c", subcore_axis_name="s")

    @functools.partial(
        pl.kernel,
        out_type=jax.ShapeDtypeStruct((nblk * dim, 128), jnp.float32),
        mesh=mesh,
        scratch_types=[
            pltpu.VMEM((dim, 128), jnp.float32),
            pltpu.VMEM((dim, 128), jnp.float32),
        ],
        compiler_params=pltpu.CompilerParams(
            use_tc_tiling_on_sc=True,
            needs_layout_passes=False,
            disable_bounds_checks=True,
        ),
    )
    def kern(tok_t_hbm, out_hbm, in_v, blk_v):
        wid = lax.axis_index("s") * NC + lax.axis_index("c")
        lanes = lax.iota(jnp.int32, LANES)
        fvec = [lanes + (h * LANES) for h in range(dim // LANES)]
        my_n = (nblk - wid + NW - 1) // NW

        def do_block(g, carry):
            b = g * NW + wid
            pltpu.sync_copy(tok_t_hbm.at[:, pl.ds(b * 128, 128)], in_v)
            for m in range(vregs):
                tl = m // 2
                tt = jnp.full((LANES,), tl, jnp.int32)
                vals = plsc.load_gather(in_v, [fvec[m % 2], tt])
                blk_v[m // 8, pl.ds((m % 8) * LANES, LANES)] = vals
            pltpu.sync_copy(blk_v, out_hbm.at[pl.ds(b * dim, dim)])
            return carry

        lax.fori_loop(0, my_n, do_block, None)

    return kern


def _make_lookup_kernel(batch, maxlen, dim, vrows, tchunk):
    bw = batch // NW
    n_chunks = maxlen // tchunk
    groups = bw // LANES

    mesh = plsc.VectorSubcoreMesh(core_axis_name="c", subcore_axis_name="s")

    @functools.partial(
        pl.kernel,
        out_type=jax.ShapeDtypeStruct((maxlen, dim, batch), jnp.float32),
        mesh=mesh,
        scratch_types=[
            pltpu.VMEM((tchunk, bw), jnp.int32),
            pltpu.VMEM((tchunk, bw, dim), jnp.float32),
            pltpu.VMEM((tchunk, dim, bw), jnp.float32),
            pltpu.VMEM((tchunk, dim, LANES), jnp.float32),
            pltpu.SemaphoreType.DMA,
        ],
        compiler_params=pltpu.CompilerParams(
            use_tc_tiling_on_sc=False, needs_layout_passes=False
        ),
    )
    def kern(x_hbm, tok_hbm, pos_hbm, out_hbm, idx_v, rows_v, blk_v, pos_v, sem):
        wid = lax.axis_index("s") * NC + lax.axis_index("c")
        b0 = wid * bw
        lanes = lax.iota(jnp.int32, LANES)
        rowq = [lanes + (q * LANES) for q in range(groups)]

        def do_chunk(g, carry):
            t0 = g * tchunk
            pltpu.sync_copy(pos_hbm.at[pl.ds(t0, tchunk)], pos_v)
            pltpu.sync_copy(x_hbm.at[pl.ds(t0, tchunk), pl.ds(b0, bw)], idx_v)
            for j in range(tchunk):
                pltpu.async_copy(tok_hbm.at[idx_v.at[j]], rows_v.at[j], sem)
            for j in range(tchunk):
                pltpu.make_async_copy(
                    tok_hbm.at[idx_v.at[j]], rows_v.at[j], sem
                ).wait()

            @plsc.parallel_loop(0, tchunk, 1)
            def transpose_add(j):
                src = rows_v.at[j]
                dst = blk_v.at[j]
                pos_j = pos_v.at[j]
                for f in range(dim):
                    pv = pos_j[f, pl.ds(0, LANES)]
                    ff = jnp.full((LANES,), f, jnp.int32)
                    vals = [
                        plsc.load_gather(src, [rowq[q], ff])
                        for q in range(groups)
                    ]
                    for q in range(groups):
                        dst[f, pl.ds(q * LANES, LANES)] = vals[q] + pv

            pltpu.sync_copy(
                blk_v, out_hbm.at[pl.ds(t0, tchunk), :, pl.ds(b0, bw)]
            )
            return carry

        lax.fori_loop(0, n_chunks, do_chunk, None)

    return kern


@jax.jit
def kernel(x, token_emb, pos_emb):
    batch, maxlen = x.shape
    vocab, dim = token_emb.shape
    nblk = -(-vocab // 128)
    vrows = nblk * 128

    relayout = _make_relayout_kernel(dim, vocab)
    tok_rows = relayout(token_emb.T)
    tok_lin = tok_rows.reshape(vrows, dim)

    xt = x.T.astype(jnp.int32)
    posb = jnp.broadcast_to(pos_emb[:, :, None], (maxlen, dim, LANES))
    lookup = _make_lookup_kernel(batch, maxlen, dim, vrows, tchunk=8)
    out_t = lookup(xt, tok_lin, posb)
    return out_t.transpose(2, 0, 1)


# speedup vs baseline: 1.1660x; 1.1660x over previous
import functools

import jax
import jax.numpy as jnp
from jax import lax
from jax.experimental import pallas as pl
from jax.experimental.pallas import tpu as pltpu
from jax.experimental.pallas import tpu_sc as plsc

NC = 2
NS = 16
LANES = 16
NW = NC * NS


def _make_relayout_kernel(dim, vocab):
    nblk = -(-vocab // 128)
    vregs = 128 * dim // LANES

    mesh = plsc.VectorSubcoreMesh(core_axis_name="c", subcore_axis_name="s")

    DEPTH = 4

    @functools.partial(
        pl.kernel,
        out_type=jax.ShapeDtypeStruct((nblk * dim, 128), jnp.float32),
        mesh=mesh,
        scratch_types=[
            pltpu.VMEM((DEPTH, dim, 128), jnp.float32),
            pltpu.VMEM((DEPTH, dim, 128), jnp.float32),
            pltpu.SemaphoreType.DMA((DEPTH,)),
            pltpu.SemaphoreType.DMA((DEPTH,)),
        ],
        compiler_params=pltpu.CompilerParams(
            use_tc_tiling_on_sc=True,
            needs_layout_passes=False,
            disable_bounds_checks=True,
        ),
    )
    def kern(tok_t_hbm, out_hbm, in_v, blk_v, gsem, ssem):
        wid = lax.axis_index("s") * NC + lax.axis_index("c")
        lanes = lax.iota(jnp.int32, LANES)
        fvec = [lanes + (h * LANES) for h in range(dim // LANES)]
        my_n = (nblk - wid + NW - 1) // NW

        def in_copy(g, s):
            b = g * NW + wid
            return pltpu.make_async_copy(
                tok_t_hbm.at[:, pl.ds(b * 128, 128)], in_v.at[s], gsem.at[s]
            )

        def out_copy(g, s):
            b = g * NW + wid
            return pltpu.make_async_copy(
                blk_v.at[s], out_hbm.at[pl.ds(b * dim, dim)], ssem.at[s]
            )

        for k in range(DEPTH):
            in_copy(k, k).start()

        def do_block(g, carry):
            s = lax.rem(g, DEPTH)
            in_copy(g, s).wait()

            @pl.when(g >= DEPTH)
            def _():
                out_copy(g - DEPTH, s).wait()

            for m in range(vregs):
                tl = m // 2
                tt = jnp.full((LANES,), tl, jnp.int32)
                vals = plsc.load_gather(in_v.at[s], [fvec[m % 2], tt])
                blk_v[s, m // 8, pl.ds((m % 8) * LANES, LANES)] = vals
            out_copy(g, s).start()

            @pl.when(g + DEPTH < my_n)
            def _():
                in_copy(g + DEPTH, s).start()

            return carry

        lax.fori_loop(0, my_n, do_block, None)

        def drain(k, carry):
            g = my_n - DEPTH + k
            out_copy(g, lax.rem(g, DEPTH)).wait()
            return carry

        lax.fori_loop(0, DEPTH, drain, None)

    return kern


def _make_lookup_kernel(batch, maxlen, dim, vrows, tchunk):
    bw = batch // NW
    n_chunks = maxlen // tchunk
    groups = bw // LANES

    mesh = plsc.VectorSubcoreMesh(core_axis_name="c", subcore_axis_name="s")

    @functools.partial(
        pl.kernel,
        out_type=jax.ShapeDtypeStruct((maxlen, dim, batch), jnp.float32),
        mesh=mesh,
        scratch_types=[
            pltpu.VMEM((tchunk, bw), jnp.int32),
            pltpu.VMEM((tchunk, bw, dim), jnp.float32),
            pltpu.VMEM((tchunk, dim, bw), jnp.float32),
            pltpu.VMEM((tchunk, dim, LANES), jnp.float32),
            pltpu.SemaphoreType.DMA,
        ],
        compiler_params=pltpu.CompilerParams(
            use_tc_tiling_on_sc=False, needs_layout_passes=False
        ),
    )
    def kern(x_hbm, tok_hbm, pos_hbm, out_hbm, idx_v, rows_v, blk_v, pos_v, sem):
        wid = lax.axis_index("s") * NC + lax.axis_index("c")
        b0 = wid * bw
        lanes = lax.iota(jnp.int32, LANES)
        rowq = [lanes + (q * LANES) for q in range(groups)]

        def do_chunk(g, carry):
            t0 = g * tchunk
            pltpu.sync_copy(pos_hbm.at[pl.ds(t0, tchunk)], pos_v)
            pltpu.sync_copy(x_hbm.at[pl.ds(t0, tchunk), pl.ds(b0, bw)], idx_v)
            for j in range(tchunk):
                pltpu.async_copy(tok_hbm.at[idx_v.at[j]], rows_v.at[j], sem)
            for j in range(tchunk):
                pltpu.make_async_copy(
                    tok_hbm.at[idx_v.at[j]], rows_v.at[j], sem
                ).wait()

            @plsc.parallel_loop(0, tchunk, 1)
            def transpose_add(j):
                src = rows_v.at[j]
                dst = blk_v.at[j]
                pos_j = pos_v.at[j]
                for f in range(dim):
                    pv = pos_j[f, pl.ds(0, LANES)]
                    ff = jnp.full((LANES,), f, jnp.int32)
                    vals = [
                        plsc.load_gather(src, [rowq[q], ff])
                        for q in range(groups)
                    ]
                    for q in range(groups):
                        dst[f, pl.ds(q * LANES, LANES)] = vals[q] + pv

            pltpu.sync_copy(
                blk_v, out_hbm.at[pl.ds(t0, tchunk), :, pl.ds(b0, bw)]
            )
            return carry

        lax.fori_loop(0, n_chunks, do_chunk, None)

    return kern


@jax.jit
def kernel(x, token_emb, pos_emb):
    batch, maxlen = x.shape
    vocab, dim = token_emb.shape
    nblk = -(-vocab // 128)
    vrows = nblk * 128

    relayout = _make_relayout_kernel(dim, vocab)
    tok_rows = relayout(token_emb.T)
    tok_lin = tok_rows.reshape(vrows, dim)

    xt = x.T.astype(jnp.int32)
    posb = jnp.broadcast_to(pos_emb[:, :, None], (maxlen, dim, LANES))
    lookup = _make_lookup_kernel(batch, maxlen, dim, vrows, tchunk=8)
    out_t = lookup(xt, tok_lin, posb)
    return out_t.transpose(2, 0, 1)


# speedup vs baseline: 1.3647x; 1.1704x over previous
import functools

import jax
import jax.numpy as jnp
from jax import lax
from jax.experimental import pallas as pl
from jax.experimental.pallas import tpu as pltpu
from jax.experimental.pallas import tpu_sc as plsc

NC = 2
NS = 16
LANES = 16
NW = NC * NS


def _make_relayout_kernel(dim, vocab):
    nblk = -(-vocab // 128)
    vregs = 128 * dim // LANES

    mesh = plsc.VectorSubcoreMesh(core_axis_name="c", subcore_axis_name="s")

    DEPTH = 4

    @functools.partial(
        pl.kernel,
        out_type=jax.ShapeDtypeStruct((nblk * dim, 128), jnp.float32),
        mesh=mesh,
        scratch_types=[
            pltpu.VMEM((DEPTH, dim, 128), jnp.float32),
            pltpu.VMEM((DEPTH, dim, 128), jnp.float32),
            pltpu.SemaphoreType.DMA((DEPTH,)),
            pltpu.SemaphoreType.DMA((DEPTH,)),
        ],
        compiler_params=pltpu.CompilerParams(
            use_tc_tiling_on_sc=True,
            needs_layout_passes=False,
            disable_bounds_checks=True,
        ),
    )
    def kern(tok_t_hbm, out_hbm, in_v, blk_v, gsem, ssem):
        wid = lax.axis_index("s") * NC + lax.axis_index("c")
        lanes = lax.iota(jnp.int32, LANES)
        fvec = [lanes + (h * LANES) for h in range(dim // LANES)]
        my_n = (nblk - wid + NW - 1) // NW

        def in_copy(g, s):
            b = g * NW + wid
            return pltpu.make_async_copy(
                tok_t_hbm.at[:, pl.ds(b * 128, 128)], in_v.at[s], gsem.at[s]
            )

        def out_copy(g, s):
            b = g * NW + wid
            return pltpu.make_async_copy(
                blk_v.at[s], out_hbm.at[pl.ds(b * dim, dim)], ssem.at[s]
            )

        for k in range(DEPTH):
            in_copy(k, k).start()

        def do_block(g, carry):
            s = lax.rem(g, DEPTH)
            in_copy(g, s).wait()

            @pl.when(g >= DEPTH)
            def _():
                out_copy(g - DEPTH, s).wait()

            for m0 in range(0, vregs, 8):
                vals = []
                for m in range(m0, m0 + 8):
                    tt = jnp.full((LANES,), m // 2, jnp.int32)
                    vals.append(
                        plsc.load_gather(in_v.at[s], [fvec[m % 2], tt])
                    )
                for i, m in enumerate(range(m0, m0 + 8)):
                    blk_v[s, m // 8, pl.ds((m % 8) * LANES, LANES)] = vals[i]
            out_copy(g, s).start()

            @pl.when(g + DEPTH < my_n)
            def _():
                in_copy(g + DEPTH, s).start()

            return carry

        lax.fori_loop(0, my_n, do_block, None)

        def drain(k, carry):
            g = my_n - DEPTH + k
            out_copy(g, lax.rem(g, DEPTH)).wait()
            return carry

        lax.fori_loop(0, DEPTH, drain, None)

    return kern


def _make_lookup_kernel(batch, maxlen, dim, vrows, tchunk):
    bw = batch // NW
    n_chunks = maxlen // tchunk
    groups = bw // LANES

    mesh = plsc.VectorSubcoreMesh(core_axis_name="c", subcore_axis_name="s")

    @functools.partial(
        pl.kernel,
        out_type=jax.ShapeDtypeStruct((maxlen, dim, batch), jnp.float32),
        mesh=mesh,
        scratch_types=[
            pltpu.VMEM((tchunk, bw), jnp.int32),
            pltpu.VMEM((tchunk, bw, dim), jnp.float32),
            pltpu.VMEM((tchunk, dim, bw), jnp.float32),
            pltpu.VMEM((tchunk, dim, LANES), jnp.float32),
            pltpu.SemaphoreType.DMA,
        ],
        compiler_params=pltpu.CompilerParams(
            use_tc_tiling_on_sc=False, needs_layout_passes=False
        ),
    )
    def kern(x_hbm, tok_hbm, pos_hbm, out_hbm, idx_v, rows_v, blk_v, pos_v, sem):
        wid = lax.axis_index("s") * NC + lax.axis_index("c")
        b0 = wid * bw
        lanes = lax.iota(jnp.int32, LANES)
        rowq = [lanes + (q * LANES) for q in range(groups)]

        def do_chunk(g, carry):
            t0 = g * tchunk
            pltpu.sync_copy(pos_hbm.at[pl.ds(t0, tchunk)], pos_v)
            pltpu.sync_copy(x_hbm.at[pl.ds(t0, tchunk), pl.ds(b0, bw)], idx_v)
            for j in range(tchunk):
                pltpu.async_copy(tok_hbm.at[idx_v.at[j]], rows_v.at[j], sem)
            for j in range(tchunk):
                pltpu.make_async_copy(
                    tok_hbm.at[idx_v.at[j]], rows_v.at[j], sem
                ).wait()

            @plsc.parallel_loop(0, tchunk, 1)
            def transpose_add(j):
                src = rows_v.at[j]
                dst = blk_v.at[j]
                pos_j = pos_v.at[j]
                for f in range(dim):
                    pv = pos_j[f, pl.ds(0, LANES)]
                    ff = jnp.full((LANES,), f, jnp.int32)
                    vals = [
                        plsc.load_gather(src, [rowq[q], ff])
                        for q in range(groups)
                    ]
                    for q in range(groups):
                        dst[f, pl.ds(q * LANES, LANES)] = vals[q] + pv

            pltpu.sync_copy(
                blk_v, out_hbm.at[pl.ds(t0, tchunk), :, pl.ds(b0, bw)]
            )
            return carry

        lax.fori_loop(0, n_chunks, do_chunk, None)

    return kern


@jax.jit
def kernel(x, token_emb, pos_emb):
    batch, maxlen = x.shape
    vocab, dim = token_emb.shape
    nblk = -(-vocab // 128)
    vrows = nblk * 128

    relayout = _make_relayout_kernel(dim, vocab)
    tok_rows = relayout(token_emb.T)
    tok_lin = tok_rows.reshape(vrows, dim)

    xt = x.T.astype(jnp.int32)
    posb = jnp.broadcast_to(pos_emb[:, :, None], (maxlen, dim, LANES))
    lookup = _make_lookup_kernel(batch, maxlen, dim, vrows, tchunk=8)
    out_t = lookup(xt, tok_lin, posb)
    return out_t.transpose(2, 0, 1)


# speedup vs baseline: 1.3659x; 1.0009x over previous
import functools

import jax
import jax.numpy as jnp
from jax import lax
from jax.experimental import pallas as pl
from jax.experimental.pallas import tpu as pltpu
from jax.experimental.pallas import tpu_sc as plsc

NC = 2
NS = 16
LANES = 16
NW = NC * NS


def _make_relayout_kernel(dim, vocab):
    nblk = -(-vocab // 128)
    vregs = 128 * dim // LANES

    mesh = plsc.VectorSubcoreMesh(core_axis_name="c", subcore_axis_name="s")

    DEPTH = 4

    @functools.partial(
        pl.kernel,
        out_type=jax.ShapeDtypeStruct((nblk * dim, 128), jnp.float32),
        mesh=mesh,
        scratch_types=[
            pltpu.VMEM((DEPTH, dim, 128), jnp.float32),
            pltpu.VMEM((DEPTH, dim, 128), jnp.float32),
            pltpu.SemaphoreType.DMA((DEPTH,)),
            pltpu.SemaphoreType.DMA((DEPTH,)),
        ],
        compiler_params=pltpu.CompilerParams(
            use_tc_tiling_on_sc=True,
            needs_layout_passes=False,
            disable_bounds_checks=True,
        ),
    )
    def kern(tok_t_hbm, out_hbm, in_v, blk_v, gsem, ssem):
        wid = lax.axis_index("s") * NC + lax.axis_index("c")
        lanes = lax.iota(jnp.int32, LANES)
        fvec = [lanes + (h * LANES) for h in range(dim // LANES)]
        my_n = (nblk - wid + NW - 1) // NW

        def in_copy(g, s):
            b = g * NW + wid
            return pltpu.make_async_copy(
                tok_t_hbm.at[:, pl.ds(b * 128, 128)], in_v.at[s], gsem.at[s]
            )

        def out_copy(g, s):
            b = g * NW + wid
            return pltpu.make_async_copy(
                blk_v.at[s], out_hbm.at[pl.ds(b * dim, dim)], ssem.at[s]
            )

        for k in range(DEPTH):
            in_copy(k, k).start()

        def do_block(g, carry):
            s = lax.rem(g, DEPTH)
            in_copy(g, s).wait()

            @pl.when(g >= DEPTH)
            def _():
                out_copy(g - DEPTH, s).wait()

            for m0 in range(0, vregs, 8):
                vals = []
                for m in range(m0, m0 + 8):
                    tt = jnp.full((LANES,), m // 2, jnp.int32)
                    vals.append(
                        plsc.load_gather(in_v.at[s], [fvec[m % 2], tt])
                    )
                for i, m in enumerate(range(m0, m0 + 8)):
                    blk_v[s, m // 8, pl.ds((m % 8) * LANES, LANES)] = vals[i]
            out_copy(g, s).start()

            @pl.when(g + DEPTH < my_n)
            def _():
                in_copy(g + DEPTH, s).start()

            return carry

        lax.fori_loop(0, my_n, do_block, None)

        def drain(k, carry):
            g = my_n - DEPTH + k
            out_copy(g, lax.rem(g, DEPTH)).wait()
            return carry

        lax.fori_loop(0, DEPTH, drain, None)

    return kern


def _make_lookup_kernel(batch, maxlen, dim, vrows, tchunk):
    bw = batch // NW
    n_chunks = maxlen // tchunk
    groups = bw // LANES

    mesh = plsc.VectorSubcoreMesh(core_axis_name="c", subcore_axis_name="s")

    @functools.partial(
        pl.kernel,
        out_type=jax.ShapeDtypeStruct((maxlen, dim, batch), jnp.float32),
        mesh=mesh,
        scratch_types=[
            pltpu.VMEM((tchunk * bw,), jnp.int32),
            pltpu.VMEM((tchunk * bw, dim), jnp.float32),
            pltpu.VMEM((tchunk, dim, bw), jnp.float32),
            pltpu.VMEM((tchunk, dim, LANES), jnp.float32),
            pltpu.SemaphoreType.DMA,
        ],
        compiler_params=pltpu.CompilerParams(
            use_tc_tiling_on_sc=False, needs_layout_passes=False
        ),
    )
    def kern(x_hbm, tok_hbm, pos_hbm, out_hbm, idx_v, rows_v, blk_v, pos_v, sem):
        wid = lax.axis_index("s") * NC + lax.axis_index("c")
        b0 = wid * bw
        lanes = lax.iota(jnp.int32, LANES)
        rowq = [lanes + (q * LANES) for q in range(groups)]

        def do_chunk(g, carry):
            t0 = g * tchunk
            pltpu.sync_copy(pos_hbm.at[pl.ds(t0, tchunk)], pos_v)
            pltpu.sync_copy(x_hbm.at[g, wid], idx_v)
            pltpu.async_copy(tok_hbm.at[idx_v], rows_v, sem).wait()

            @plsc.parallel_loop(0, tchunk, 1)
            def transpose_add(j):
                src = rows_v.at[pl.ds(j * bw, bw)]
                dst = blk_v.at[j]
                pos_j = pos_v.at[j]
                for f in range(dim):
                    pv = pos_j[f, pl.ds(0, LANES)]
                    ff = jnp.full((LANES,), f, jnp.int32)
                    vals = [
                        plsc.load_gather(src, [rowq[q], ff])
                        for q in range(groups)
                    ]
                    for q in range(groups):
                        dst[f, pl.ds(q * LANES, LANES)] = vals[q] + pv

            pltpu.sync_copy(
                blk_v, out_hbm.at[pl.ds(t0, tchunk), :, pl.ds(b0, bw)]
            )
            return carry

        lax.fori_loop(0, n_chunks, do_chunk, None)

    return kern


@jax.jit
def kernel(x, token_emb, pos_emb):
    batch, maxlen = x.shape
    vocab, dim = token_emb.shape
    nblk = -(-vocab // 128)
    vrows = nblk * 128

    relayout = _make_relayout_kernel(dim, vocab)
    tok_rows = relayout(token_emb.T)
    tok_lin = tok_rows.reshape(vrows, dim)

    tchunk, bw = 8, batch // NW
    xt = (
        x.T.astype(jnp.int32)
        .reshape(maxlen // tchunk, tchunk, NW, bw)
        .transpose(0, 2, 1, 3)
        .reshape(maxlen // tchunk, NW, tchunk * bw)
    )
    posb = jnp.broadcast_to(pos_emb[:, :, None], (maxlen, dim, LANES))
    lookup = _make_lookup_kernel(batch, maxlen, dim, vrows, tchunk=tchunk)
    out_t = lookup(xt, tok_lin, posb)
    return out_t.transpose(2, 0, 1)


# speedup vs baseline: 1.7083x; 1.2507x over previous
import functools

import jax
import jax.numpy as jnp
from jax import lax
from jax.experimental import pallas as pl
from jax.experimental.pallas import tpu as pltpu
from jax.experimental.pallas import tpu_sc as plsc

NC = 2
NS = 16
LANES = 16
NW = NC * NS


def _make_sc_kernel(batch, maxlen, dim, nseq):
    seq_per_w = batch // NW
    n_chunks = seq_per_w // nseq
    half = dim // LANES

    mesh = plsc.VectorSubcoreMesh(core_axis_name="c", subcore_axis_name="s")

    @functools.partial(
        pl.kernel,
        out_type=jax.ShapeDtypeStruct((batch, maxlen, dim), jnp.float32),
        mesh=mesh,
        scratch_types=[
            pltpu.VMEM((nseq, maxlen), jnp.int32),
            pltpu.VMEM((nseq, maxlen, dim), jnp.float32),
            pltpu.VMEM((maxlen, dim), jnp.float32),
            pltpu.SemaphoreType.DMA,
        ],
        compiler_params=pltpu.CompilerParams(use_tc_tiling_on_sc=False),
    )
    def kern(x_hbm, tok_hbm, pos_hbm, out_hbm, idx_v, rows_v, pos_v, sem):
        wid = lax.axis_index("s") * NC + lax.axis_index("c")
        seq_base = wid * seq_per_w

        pltpu.sync_copy(pos_hbm, pos_v)

        def do_chunk(g, carry):
            s0 = seq_base + g * nseq
            pltpu.sync_copy(x_hbm.at[pl.ds(s0, nseq)], idx_v)
            for j in range(nseq):
                pltpu.async_copy(tok_hbm.at[idx_v.at[j]], rows_v.at[j], sem)
            for j in range(nseq):
                pltpu.make_async_copy(
                    tok_hbm.at[idx_v.at[j]], rows_v.at[j], sem
                ).wait()

            def add_pos(j, c2):
                for h in range(half):
                    pv = pos_v[j, pl.ds(h * LANES, LANES)]
                    for s in range(nseq):
                        plsc.addupdate(
                            rows_v.at[s, j, pl.ds(h * LANES, LANES)], pv
                        )
                return c2

            lax.fori_loop(0, maxlen, add_pos, None)
            pltpu.sync_copy(rows_v, out_hbm.at[pl.ds(s0, nseq)])
            return carry

        lax.fori_loop(0, n_chunks, do_chunk, None)

    return kern


@jax.jit
def kernel(x, token_emb, pos_emb):
    batch, maxlen = x.shape
    dim = token_emb.shape[1]
    k = _make_sc_kernel(batch, maxlen, dim, nseq=8)
    return k(x.astype(jnp.int32), token_emb, pos_emb)
